# Initial kernel scaffold; baseline (speedup 1.0000x reference)
import jax
import jax.numpy as jnp
from jax.experimental import pallas as pl


def kernel(x, edge_index, params):
    raise NotImplementedError("write your pallas kernel here")


# speedup vs baseline: 7.3124x; 7.3124x over previous
import functools
import math

import jax
import jax.numpy as jnp
from jax import lax
from jax.experimental import pallas as pl
from jax.experimental.pallas import tpu as pltpu
from jax.experimental.pallas import tpu_sc as plsc

EPS = 1e-5
IMAX = 2**31 - 1


def _fold(p, cin):
    s = math.sqrt(1.0 + EPS)
    w1a, w1b = p["w1"][:, :cin], p["w1"][:, cin:]
    g1s = p["g1"] / s
    g2s = p["g2"] / s
    return dict(
        wU=(w1a - w1b).T,
        wV=w1b.T,
        bU=p["b1"][None, :],
        w2=(p["w2"] * g1s[None, :]).T,
        b2=(p["b2"] + p["be1"] @ p["w2"].T)[None, :],
        w3=(p["w3"] * g2s[None, :]).T,
        b3=(p["b3"] + p["be2"] @ p["w3"].T)[None, :],
        a3=(p["g3"] / s)[None, :],
        be3=p["be3"][None, :],
        wS=(p["ws"] * (p["gs"] / s)[:, None]).T,
        bS=(p["gs"] * p["bs"] / s + p["bes"])[None, :],
    )


def _prep0_body(x_ref, gw_ref, gb_ref, gms_ref, wU_ref, bU_ref, wV_ref,
                wS_ref, bS_ref, t_ref, s_ref):
    x = x_ref[...]
    mean = jnp.mean(x, axis=0, keepdims=True)
    o = x - gms_ref[...] * mean
    var = jnp.mean(o * o, axis=0, keepdims=True)
    h = gw_ref[...] * o * lax.rsqrt(var + EPS) + gb_ref[...]
    u = jnp.dot(h, wU_ref[...], preferred_element_type=jnp.float32) + bU_ref[...]
    v = jnp.dot(h, wV_ref[...], preferred_element_type=jnp.float32)
    t_ref[...] = jnp.concatenate([u, v], axis=1)
    s_ref[...] = jnp.dot(h, wS_ref[...], preferred_element_type=jnp.float32) + bS_ref[...]


def _prep0(x, p, f):
    n = x.shape[0]
    co = f["wU"].shape[1]
    outs = [jax.ShapeDtypeStruct((n, 2 * co), jnp.float32),
            jax.ShapeDtypeStruct((n, co), jnp.float32)]
    return pl.pallas_call(_prep0_body, out_shape=outs)(
        x, p["gn_w"][None, :], p["gn_b"][None, :], p["gn_ms"][None, :],
        f["wU"], f["bU"], f["wV"], f["wS"], f["bS"])


def _sc_gather(table, idx):
    e = idx.shape[0]
    d = table.shape[1]
    C = 80
    nch = e // C
    NW = 32
    per_w = (nch + NW - 1) // NW
    mesh = plsc.VectorSubcoreMesh(core_axis_name="c", subcore_axis_name="s")

    @functools.partial(
        pl.kernel, mesh=mesh,
        out_type=jax.ShapeDtypeStruct((e, d), jnp.float32),
        scratch_types=[
            pltpu.VMEM((C,), jnp.int32),
            pltpu.VMEM((C, d), jnp.float32),
            pltpu.SemaphoreType.DMA,
        ],
    )
    def k(table_hbm, idx_hbm, out_hbm, idx_v, rows_v, sem):
        wid = lax.axis_index("s") * 2 + lax.axis_index("c")

        def body(t, carry):
            c = wid + NW * t

            @pl.when(c < nch)
            def _():
                base = pl.multiple_of(c * C, 8)
                pltpu.sync_copy(idx_hbm.at[pl.ds(base, C)], idx_v)
                pltpu.async_copy(table_hbm.at[idx_v], rows_v, sem).wait()
                pltpu.sync_copy(rows_v, out_hbm.at[pl.ds(base, C)])

            return carry

        lax.fori_loop(0, per_w, body, 0)

    return k(table, idx)


def _edge_mlp_body(gd_ref, gs_ref, w2_ref, b2_ref, w3_ref, b3_ref, out_ref, *, co):
    z1 = jnp.maximum(gd_ref[:, :co] + gs_ref[:, co:], 0.0)
    z2 = jnp.maximum(
        jnp.dot(z1, w2_ref[...], preferred_element_type=jnp.float32) + b2_ref[...], 0.0)
    z3 = jnp.maximum(
        jnp.dot(z2, w3_ref[...], preferred_element_type=jnp.float32) + b3_ref[...], 0.0)
    out_ref[...] = jnp.concatenate([z3, jnp.ones_like(z3)], axis=1)


def _edge_mlp(gd, gs, f):
    e = gd.shape[0]
    co = f["w2"].shape[0]
    BE = 2000
    grid = (e // BE,)
    bspec = pl.BlockSpec((BE, 2 * co), lambda i: (i, 0))
    wspec = pl.BlockSpec((co, co), lambda i: (0, 0))
    bias = pl.BlockSpec((1, co), lambda i: (0, 0))
    return pl.pallas_call(
        functools.partial(_edge_mlp_body, co=co),
        grid=grid,
        in_specs=[bspec, bspec, wspec, bias, wspec, bias],
        out_specs=bspec,
        out_shape=jax.ShapeDtypeStruct((e, 2 * co), jnp.float32),
    )(gd, gs, f["w2"], f["b2"], f["w3"], f["b3"])


def _sc_scatter(z3, dst, n):
    e, d = z3.shape
    C = 80
    nch = e // C
    NW = 32
    NS = 16
    per_w = (nch + NW - 1) // NW
    rps = ((n + NS - 1) // NS + 7) // 8 * 8
    last = n - (NS - 1) * rps
    mesh = plsc.VectorSubcoreMesh(core_axis_name="c", subcore_axis_name="s")

    zero_d = jnp.zeros((n, d), jnp.float32)

    @functools.partial(
        pl.kernel, mesh=mesh,
        out_type=jax.ShapeDtypeStruct((2, n, d), jnp.float32),
        scratch_types=[
            pltpu.VMEM((C,), jnp.int32),
            pltpu.VMEM((C, d), jnp.float32),
            pltpu.VMEM_SHARED((n, d), jnp.float32),
        ],
    )
    def k(z3_hbm, dst_hbm, zd_hbm, outs_hbm, idx_v, val_v, acc_sh):
        cid = lax.axis_index("c")
        sid = lax.axis_index("s")
        wid = sid * 2 + cid
        r0 = pl.multiple_of(sid * rps, 8)

        @pl.when(sid < NS - 1)
        def _():
            pltpu.sync_copy(zd_hbm.at[pl.ds(r0, rps)], acc_sh.at[pl.ds(r0, rps)])

        @pl.when(sid == NS - 1)
        def _():
            pltpu.sync_copy(zd_hbm.at[pl.ds(r0, last)], acc_sh.at[pl.ds(r0, last)])

        plsc.subcore_barrier()

        def body(t, carry):
            c = wid + NW * t

            @pl.when(c < nch)
            def _():
                base = pl.multiple_of(c * C, 8)
                pltpu.sync_copy(dst_hbm.at[pl.ds(base, C)], idx_v)
                pltpu.sync_copy(z3_hbm.at[pl.ds(base, C)], val_v)
                pltpu.sync_copy(val_v, acc_sh.at[idx_v], add=True)

            return carry

        lax.fori_loop(0, per_w, body, 0)
        plsc.subcore_barrier()

        @pl.when(sid < NS - 1)
        def _():
            pltpu.sync_copy(acc_sh.at[pl.ds(r0, rps)], outs_hbm.at[cid, pl.ds(r0, rps)])

        @pl.when(sid == NS - 1)
        def _():
            pltpu.sync_copy(acc_sh.at[pl.ds(r0, last)], outs_hbm.at[cid, pl.ds(r0, last)])

    return k(z3, dst, zero_d)


def _combine_body(sacc_ref, sc_ref, a3_ref, be3_ref,
                  wU_ref, bU_ref, wV_ref, wS_ref, bS_ref,
                  h_ref, u_ref, v_ref, s_ref, *, co):
    full = sacc_ref[0] + sacc_ref[1]
    S = full[:, :co]
    c = full[:, co:co + 1]
    mean = S / jnp.maximum(c, 1.0)
    ec = jnp.where(c > 0.0, a3_ref[...] * mean + be3_ref[...], 0.0)
    h = ec + sc_ref[...]
    h_ref[...] = h
    u_ref[...] = jnp.dot(h, wU_ref[...], preferred_element_type=jnp.float32) + bU_ref[...]
    v_ref[...] = jnp.dot(h, wV_ref[...], preferred_element_type=jnp.float32)
    s_ref[...] = jnp.dot(h, wS_ref[...], preferred_element_type=jnp.float32) + bS_ref[...]


def _combine1(sacc, sc, f1, f2):
    n = sc.shape[0]
    ci = sc.shape[1]
    co = f2["wU"].shape[1]
    outs = [jax.ShapeDtypeStruct((n, ci), jnp.float32)] + \
           [jax.ShapeDtypeStruct((n, co), jnp.float32)] * 3
    return pl.pallas_call(functools.partial(_combine_body, co=ci), out_shape=outs)(
        sacc, sc, f1["a3"], f1["be3"],
        f2["wU"], f2["bU"], f2["wV"], f2["wS"], f2["bS"])


def _knn_body(x_ref, o_ref, s0, s1, s2, s3, *, n, br, w, nch):
    i = pl.program_id(0)
    r0 = i * br
    xr = x_ref[pl.ds(r0, br), :] * (-2.0)
    grow = r0 + lax.broadcasted_iota(jnp.int32, (br, 1), 0)
    lane = lax.broadcasted_iota(jnp.int32, (1, w), 1)
    for ref in (s0, s1, s2, s3):
        ref[...] = jnp.full((br, w), IMAX, jnp.int32)

    d = x_ref.shape[1]
    ones = jnp.ones((1, d), jnp.float32)
    dn = (((1,), (1,)), ((), ()))

    def chunk(t, carry):
        xc = x_ref[pl.ds(t * w, w), :]
        dotv = lax.dot_general(xr, xc, dn, preferred_element_type=jnp.float32)
        sqc = lax.dot_general(ones, xc * xc, dn, preferred_element_type=jnp.float32)
        e = dotv + sqc
        b = lax.bitcast_convert_type(e, jnp.int32)
        ikey = b ^ ((b >> 31) & 0x7FFFFFFF)
        key = (ikey & -32) | t
        gcol = t * w + lane
        bad = (gcol == grow) | (gcol >= n)
        key = jnp.where(bad, IMAX, key)
        cur = key
        for ref in (s0, s1, s2, s3):
            m = ref[...]
            ref[...] = jnp.minimum(m, cur)
            cur = jnp.maximum(m, cur)
        return carry

    lax.fori_loop(0, nch, chunk, 0)

    a = [s0[...], s1[...], s2[...], s3[...]]
    cols = []
    for _ in range(4):
        mm = jnp.minimum(jnp.minimum(a[0], a[1]), jnp.minimum(a[2], a[3]))
        rm = jnp.min(mm, axis=1, keepdims=True)
        pos = jnp.min(jnp.where(mm == rm, lane, IMAX), axis=1, keepdims=True)
        cols.append((rm & 31) * w + pos)
        selc = lane == pos
        a = [jnp.where(selc & (ak == rm), IMAX, ak) for ak in a]
    o_ref[...] = jnp.concatenate(cols, axis=1)


def _knn(h):
    n, d = h.shape
    W = 512
    npad = ((n + W - 1) // W) * W
    nch = npad // W
    BR = 1000
    grid = (n // BR,)
    xpad = jnp.pad(h, ((0, npad - n), (0, 0)))
    body = functools.partial(_knn_body, n=n, br=BR, w=W, nch=nch)
    return pl.pallas_call(
        body,
        grid=grid,
        in_specs=[pl.BlockSpec((npad, d), lambda i: (0, 0))],
        out_specs=pl.BlockSpec((BR, 4), lambda i: (i, 0)),
        out_shape=jax.ShapeDtypeStruct((n, 4), jnp.int32),
        scratch_shapes=[pltpu.VMEM((BR, W), jnp.int32)] * 4,
    )(xpad)


def _dyn_knn_body(g_ref, u_ref, sc_ref, w2_ref, b2_ref, w3_ref, b3_ref,
                  a3_ref, be3_ref, *rest, emit_prep, bn, co):
    if emit_prep:
        wU_ref, bU_ref, wV_ref, wS_ref, bS_ref, h_ref, u2_ref, v2_ref, s2_ref = rest
    else:
        (h_ref,) = rest
    g = g_ref[...]
    u = u_ref[...]
    z1 = jnp.maximum(g + u[:, None, :], 0.0).reshape(4 * bn, co)
    z2 = jnp.maximum(
        jnp.dot(z1, w2_ref[...], preferred_element_type=jnp.float32) + b2_ref[...], 0.0)
    z3 = jnp.maximum(
        jnp.dot(z2, w3_ref[...], preferred_element_type=jnp.float32) + b3_ref[...], 0.0)
    z3r = z3.reshape(bn, 4, co)
    msum = z3r[:, 0] + z3r[:, 1] + z3r[:, 2] + z3r[:, 3]
    h = a3_ref[...] * (msum * 0.25) + be3_ref[...] + sc_ref[...]
    h_ref[...] = h
    if emit_prep:
        u2_ref[...] = jnp.dot(h, wU_ref[...], preferred_element_type=jnp.float32) + bU_ref[...]
        v2_ref[...] = jnp.dot(h, wV_ref[...], preferred_element_type=jnp.float32)
        s2_ref[...] = jnp.dot(h, wS_ref[...], preferred_element_type=jnp.float32) + bS_ref[...]


def _dyn_knn(g, u, sc, f, fnext):
    n, co = u.shape
    BN = 2000
    grid = (n // BN,)
    g3d = g.reshape(n, 4, co)
    nspec = pl.BlockSpec((BN, co), lambda i: (i, 0))
    wspec = pl.BlockSpec((co, co), lambda i: (0, 0))
    bias = pl.BlockSpec((1, co), lambda i: (0, 0))
    emit_prep = fnext is not None
    in_specs = [pl.BlockSpec((BN, 4, co), lambda i: (i, 0, 0)), nspec, nspec,
                wspec, bias, wspec, bias, bias, bias]
    args = [g3d, u, sc, f["w2"], f["b2"], f["w3"], f["b3"], f["a3"], f["be3"]]
    out_specs = [nspec]
    out_shape = [jax.ShapeDtypeStruct((n, co), jnp.float32)]
    if emit_prep:
        cn = fnext["wU"].shape[1]
        wn = pl.BlockSpec((co, cn), lambda i: (0, 0))
        bn_ = pl.BlockSpec((1, cn), lambda i: (0, 0))
        nn = pl.BlockSpec((BN, cn), lambda i: (i, 0))
        in_specs += [wn, bn_, wn, wn, bn_]
        args += [fnext["wU"], fnext["bU"], fnext["wV"], fnext["wS"], fnext["bS"]]
        out_specs += [nn] * 3
        out_shape += [jax.ShapeDtypeStruct((n, cn), jnp.float32)] * 3
    body = functools.partial(_dyn_knn_body, emit_prep=emit_prep, bn=BN, co=co)
    return pl.pallas_call(
        body, grid=grid, in_specs=in_specs, out_specs=out_specs,
        out_shape=out_shape,
    )(*args)


def _head_body(h_ref, w1_ref, b1_ref, g1_ref, c1_ref, w2_ref, b2_ref,
               g2_ref, c2_ref, wo_ref, bo_ref, o_ref):
    s = math.sqrt(1.0 + EPS)
    g = jnp.mean(h_ref[...], axis=0, keepdims=True)
    z = jnp.maximum(jnp.dot(g, w1_ref[...], preferred_element_type=jnp.float32) + b1_ref[...], 0.0)
    z = g1_ref[...] * z / s + c1_ref[...]
    z = jnp.maximum(jnp.dot(z, w2_ref[...], preferred_element_type=jnp.float32) + b2_ref[...], 0.0)
    z = g2_ref[...] * z / s + c2_ref[...]
    o = jnp.dot(z, wo_ref[...], preferred_element_type=jnp.float32) + bo_ref[...]
    m = jnp.max(o, axis=1, keepdims=True)
    e = jnp.exp(o - m)
    sm = e / jnp.sum(e, axis=1, keepdims=True)
    o_ref[...] = jnp.pad(sm, ((0, 7), (0, 126)))


def _head(h, p):
    out = pl.pallas_call(
        _head_body,
        out_shape=jax.ShapeDtypeStruct((8, 128), jnp.float32),
    )(h, p["d1_w"].T, p["d1_b"][None, :], p["bn1_g"][None, :], p["bn1_b"][None, :],
      p["d2_w"].T, p["d2_b"][None, :], p["bn2_g"][None, :], p["bn2_b"][None, :],
      p["out_w"].T, p["out_b"][None, :])
    return out[:1, :2]


def kernel(x, edge_index, params):
    p = params
    f1 = _fold(p["c1"], x.shape[1])
    f2 = _fold(p["c2"], f1["wU"].shape[1])
    f3 = _fold(p["c3"], f2["wU"].shape[1])
    n = x.shape[0]
    src, dst = edge_index[0], edge_index[1]

    t1, s1 = _prep0(x, p, f1)
    gd = _sc_gather(t1, dst)
    gs = _sc_gather(t1, src)
    z3 = _edge_mlp(gd, gs, f1)
    sacc = _sc_scatter(z3, dst, n)
    h1, u2, v2, s2 = _combine1(sacc, s1, f1, f2)

    idx2 = _knn(h1)
    g2 = _sc_gather(v2, idx2.reshape(-1))
    h2, u3, v3, s3 = _dyn_knn(g2, u2, s2, f2, f3)

    idx3 = _knn(h2)
    g3 = _sc_gather(v3, idx3.reshape(-1))
    (h3,) = _dyn_knn(g3, u3, s3, f3, None)

    return _head(h3, p)
